# baseline (speedup 1.0000x reference)
import functools

import jax
import jax.numpy as jnp
from jax import lax
from jax.experimental import pallas as pl
from jax.experimental.pallas import tpu as pltpu
from jax.experimental.pallas import tpu_sc as plsc

KEEP = 0.1
ROW_BLOCK = 512
NC, NS, L = 2, 16, 16
NW = NC * NS
CH = 8


def _splat(v):
    if getattr(v, "ndim", 0) == 0:
        return lax.broadcast(v, (L,))
    return v


_GDN = lax.GatherDimensionNumbers(
    offset_dims=(), collapsed_slice_dims=(0,), start_index_map=(0,))


def _take(v, idx):
    return lax.gather(v, idx[:, None], _GDN, (1,),
                      mode=lax.GatherScatterMode.PROMISE_IN_BOUNDS)


def _scan512(h, c, kk):
    ca = c[pl.ds(0, L)]
    cb = c[pl.ds(L, L)]
    rb = lax.rev(cb, (0,))
    crb = plsc.cumsum(rb)
    tb = _take(crb, jnp.full((L,), 15, jnp.int32))
    condb = crb >= kk
    anyb = jnp.any(condb)
    lb = jnp.minimum(_splat(plsc.all_reduce_ffs(condb)), 15)
    ra = lax.rev(ca, (0,))
    cra = plsc.cumsum(ra) + tb
    conda = cra >= kk
    la = jnp.minimum(_splat(plsc.all_reduce_ffs(conda)), 15)
    i1v = jnp.where(anyb, 31 - lb, 15 - la)
    pre = jnp.where(anyb, _take(crb - rb, lb), _take(cra - ra, la))
    i1s = lax.reduce_max(i1v, (0,))
    hv = h[pl.ds(i1s * L, L)]
    rv = lax.rev(hv, (0,))
    crv = plsc.cumsum(rv)
    cond = (pre + crv) >= kk
    lf = jnp.minimum(_splat(plsc.all_reduce_ffs(cond)), 15)
    b = i1v * L + 15 - lf
    pre2 = pre + _take(crv - rv, lf)
    return b, kk - pre2


def _sc_thresholds(xflat, S, D, k):
    rows_w = S // NW
    mesh = plsc.VectorSubcoreMesh(core_axis_name="c", subcore_axis_name="s")

    @functools.partial(
        pl.kernel,
        out_type=jax.ShapeDtypeStruct((S,), jnp.int32),
        mesh=mesh,
        scratch_types=[
            pltpu.VMEM((CH * D,), jnp.float32),
            pltpu.VMEM((512,), jnp.int32),
            pltpu.VMEM((32,), jnp.int32),
            pltpu.VMEM((512,), jnp.int32),
            pltpu.VMEM((32,), jnp.int32),
            pltpu.VMEM((rows_w,), jnp.int32),
        ],
        compiler_params=pltpu.CompilerParams(needs_layout_passes=False),
    )
    def body(x_hbm, thr_hbm, rows_v, h1, c1, h2, c2, thr_v):
        wid = lax.axis_index("s") * NC + lax.axis_index("c")
        base = wid * rows_w
        ones16 = jnp.ones((L,), jnp.int32)
        zeros16 = jnp.zeros((L,), jnp.int32)

        def row_body(ci, r, _):
            roff = r * D

            @plsc.parallel_loop(0, 512 // L, unroll=8)
            def _z(i):
                h1[pl.ds(i * L, L)] = zeros16
                h2[pl.ds(i * L, L)] = zeros16

            c1[pl.ds(0, L)] = zeros16
            c1[pl.ds(L, L)] = zeros16
            c2[pl.ds(0, L)] = zeros16
            c2[pl.ds(L, L)] = zeros16

            @plsc.parallel_loop(0, D // L, unroll=8)
            def _h1(j):
                v = rows_v[pl.ds(roff + L * j, L)]
                bits = (lax.bitcast_convert_type(v, jnp.int32)
                        & jnp.int32(0x7FFFFFFF))
                plsc.addupdate_scatter(h1, [bits >> 22], ones16)
                plsc.addupdate_scatter(c1, [bits >> 26], ones16)

            b1, k2 = _scan512(h1, c1, k)

            @plsc.parallel_loop(0, D // L, unroll=8)
            def _h2(j):
                v = rows_v[pl.ds(roff + L * j, L)]
                bits = (lax.bitcast_convert_type(v, jnp.int32)
                        & jnp.int32(0x7FFFFFFF))
                m = (bits >> 22) == b1
                plsc.addupdate_scatter(
                    h2, [(bits >> 13) & 511], ones16, mask=m)
                plsc.addupdate_scatter(
                    c2, [(bits >> 17) & 31], ones16, mask=m)

            b2, _ = _scan512(h2, c2, k2)
            val = (b1 << 22) | (b2 << 13)
            plsc.store_scatter(
                thr_v, [lax.broadcast(ci * CH + r, (L,))],
                val, mask=lax.iota(jnp.int32, L) == 0)
            return 0

        def chunk_body(ci, _c):
            pltpu.sync_copy(
                x_hbm.at[pl.ds((base + ci * CH) * D, CH * D)], rows_v)
            lax.fori_loop(0, CH, functools.partial(row_body, ci), 0)
            return 0

        lax.fori_loop(0, rows_w // CH, chunk_body, 0)
        pltpu.sync_copy(thr_v, thr_hbm.at[pl.ds(base, rows_w)])

    return body(xflat)


def _mask_body(x_ref, t_ref, _a_ref, o_ref):
    x = x_ref[...]
    thr = jax.lax.bitcast_convert_type(t_ref[...], jnp.float32)
    ax = jnp.abs(x)
    o_ref[...] = x * jax.nn.sigmoid(10.0 * (ax - thr))


def _tc_body(k, x_ref, o_ref):
    x = x_ref[...]
    bits = jax.lax.bitcast_convert_type(x, jnp.int32) & jnp.int32(0x7FFFFFFF)
    ones_col = jnp.ones((x.shape[1], 8), jnp.float32)
    kf = jnp.float32(k)
    nchain = 4
    rows = x.shape[0] // nchain
    chunks = [bits[i * rows:(i + 1) * rows] for i in range(nchain)]
    los = [jnp.zeros((rows, 1), jnp.int32) for _ in range(nchain)]
    for b in range(30, 12, -1):
        bit = jnp.int32(1 << b)
        cands = [lo | bit for lo in los]
        inds = [(c >= cand).astype(jnp.float32)
                for c, cand in zip(chunks, cands)]
        cnts = [jax.lax.dot_general(ind, ones_col, (((1,), (0,)), ((), ())),
                                    preferred_element_type=jnp.float32)[:, 0:1]
                for ind in inds]
        los = [jnp.where(cnt >= kf, cand, lo)
               for cnt, cand, lo in zip(cnts, cands, los)]
    lo = jnp.concatenate(los, axis=0)
    thr = jax.lax.bitcast_convert_type(lo, jnp.float32)
    ax = jax.lax.bitcast_convert_type(bits, jnp.float32)
    o_ref[...] = x * jax.nn.sigmoid(10.0 * (ax - thr))


SC_ROWS = 7680


def kernel(x):
    B, T, D = x.shape
    k = max(1, int(D * KEEP))
    R = B * T
    S = SC_ROWS
    xr = x.reshape(R, D)
    nsc = S // ROW_BLOCK
    thr_sc = _sc_thresholds(xr.reshape(-1), S, D, k)
    big = pl.pallas_call(
        functools.partial(_tc_body, k),
        grid=(R // ROW_BLOCK - nsc,),
        in_specs=[pl.BlockSpec((ROW_BLOCK, D), lambda i: (i + nsc, 0))],
        out_specs=pl.BlockSpec((ROW_BLOCK, D), lambda i: (i + nsc, 0)),
        out_shape=jax.ShapeDtypeStruct((R, D), x.dtype),
    )(xr)
    out = pl.pallas_call(
        _mask_body,
        grid=(nsc,),
        in_specs=[
            pl.BlockSpec((ROW_BLOCK, D), lambda i: (i, 0)),
            pl.BlockSpec((ROW_BLOCK, 1), lambda i: (i, 0)),
            pl.BlockSpec(memory_space=pltpu.MemorySpace.HBM),
        ],
        out_specs=pl.BlockSpec((ROW_BLOCK, D), lambda i: (i, 0)),
        out_shape=jax.ShapeDtypeStruct((R, D), x.dtype),
        input_output_aliases={2: 0},
    )(xr, thr_sc.reshape(S, 1), big)
    return out.reshape(B, T, D)


# speedup vs baseline: 1.1277x; 1.1277x over previous
import functools

import jax
import jax.numpy as jnp
from jax import lax
from jax.experimental import pallas as pl
from jax.experimental.pallas import tpu as pltpu
from jax.experimental.pallas import tpu_sc as plsc

KEEP = 0.1
ROW_BLOCK = 512
NC, NS, L = 2, 16, 16
NW = NC * NS
CH = 8


def _splat(v):
    if getattr(v, "ndim", 0) == 0:
        return lax.broadcast(v, (L,))
    return v


_GDN = lax.GatherDimensionNumbers(
    offset_dims=(), collapsed_slice_dims=(0,), start_index_map=(0,))


def _take(v, idx):
    return lax.gather(v, idx[:, None], _GDN, (1,),
                      mode=lax.GatherScatterMode.PROMISE_IN_BOUNDS)


def _scan512(h, c, kk):
    ca = c[pl.ds(0, L)]
    cb = c[pl.ds(L, L)]
    rb = lax.rev(cb, (0,))
    crb = plsc.cumsum(rb)
    tb = _take(crb, jnp.full((L,), 15, jnp.int32))
    condb = crb >= kk
    anyb = jnp.any(condb)
    lb = jnp.minimum(_splat(plsc.all_reduce_ffs(condb)), 15)
    ra = lax.rev(ca, (0,))
    cra = plsc.cumsum(ra) + tb
    conda = cra >= kk
    la = jnp.minimum(_splat(plsc.all_reduce_ffs(conda)), 15)
    i1v = jnp.where(anyb, 31 - lb, 15 - la)
    pre = jnp.where(anyb, _take(crb - rb, lb), _take(cra - ra, la))
    i1s = lax.reduce_max(i1v, (0,))
    hv = h[pl.ds(i1s * L, L)]
    rv = lax.rev(hv, (0,))
    crv = plsc.cumsum(rv)
    cond = (pre + crv) >= kk
    lf = jnp.minimum(_splat(plsc.all_reduce_ffs(cond)), 15)
    b = i1v * L + 15 - lf
    pre2 = pre + _take(crv - rv, lf)
    return b, kk - pre2


def _sc_thresholds(xflat, S, D, k):
    rows_w = S // NW
    mesh = plsc.VectorSubcoreMesh(core_axis_name="c", subcore_axis_name="s")

    @functools.partial(
        pl.kernel,
        out_type=jax.ShapeDtypeStruct((S,), jnp.int32),
        mesh=mesh,
        scratch_types=[
            pltpu.VMEM((CH * D,), jnp.float32),
            pltpu.VMEM((512,), jnp.int32),
            pltpu.VMEM((32,), jnp.int32),
            pltpu.VMEM((512,), jnp.int32),
            pltpu.VMEM((32,), jnp.int32),
            pltpu.VMEM((rows_w,), jnp.int32),
        ],
        compiler_params=pltpu.CompilerParams(needs_layout_passes=False),
    )
    def body(x_hbm, thr_hbm, rows_v, h1, c1, h2, c2, thr_v):
        wid = lax.axis_index("s") * NC + lax.axis_index("c")
        base = wid * rows_w
        ones16 = jnp.ones((L,), jnp.int32)
        zeros16 = jnp.zeros((L,), jnp.int32)

        def row_body(ci, r, _):
            roff = r * D

            @plsc.parallel_loop(0, 512 // L, unroll=8)
            def _z(i):
                h1[pl.ds(i * L, L)] = zeros16
                h2[pl.ds(i * L, L)] = zeros16

            c1[pl.ds(0, L)] = zeros16
            c1[pl.ds(L, L)] = zeros16
            c2[pl.ds(0, L)] = zeros16
            c2[pl.ds(L, L)] = zeros16

            @plsc.parallel_loop(0, D // L, unroll=8)
            def _h1(j):
                v = rows_v[pl.ds(roff + L * j, L)]
                bits = (lax.bitcast_convert_type(v, jnp.int32)
                        & jnp.int32(0x7FFFFFFF))
                plsc.addupdate_scatter(h1, [bits >> 22], ones16)
                plsc.addupdate_scatter(c1, [bits >> 26], ones16)

            b1, k2 = _scan512(h1, c1, k)

            @plsc.parallel_loop(0, D // L, unroll=8)
            def _h2(j):
                v = rows_v[pl.ds(roff + L * j, L)]
                bits = (lax.bitcast_convert_type(v, jnp.int32)
                        & jnp.int32(0x7FFFFFFF))
                m = (bits >> 22) == b1
                plsc.addupdate_scatter(
                    h2, [(bits >> 13) & 511], ones16, mask=m)
                plsc.addupdate_scatter(
                    c2, [(bits >> 17) & 31], ones16, mask=m)

            b2, _ = _scan512(h2, c2, k2)
            val = (b1 << 22) | (b2 << 13)
            plsc.store_scatter(
                thr_v, [lax.broadcast(ci * CH + r, (L,))],
                val, mask=lax.iota(jnp.int32, L) == 0)
            return 0

        def chunk_body(ci, _c):
            pltpu.sync_copy(
                x_hbm.at[pl.ds((base + ci * CH) * D, CH * D)], rows_v)
            lax.fori_loop(0, CH, functools.partial(row_body, ci), 0)
            return 0

        lax.fori_loop(0, rows_w // CH, chunk_body, 0)
        pltpu.sync_copy(thr_v, thr_hbm.at[pl.ds(base, rows_w)])

    return body(xflat)


def _mask_body(x_ref, t_ref, _a_ref, o_ref):
    x = x_ref[...]
    thr = jax.lax.bitcast_convert_type(t_ref[...], jnp.float32)
    ax = jnp.abs(x)
    o_ref[...] = x * jax.nn.sigmoid(10.0 * (ax - thr))


def _tc_body(k, x_ref, o_ref):
    x = x_ref[...]
    bits = jax.lax.bitcast_convert_type(x, jnp.int32) & jnp.int32(0x7FFFFFFF)
    ones_col = jnp.ones((x.shape[1], 8), jnp.float32)
    kf = jnp.float32(k)
    nchain = 4
    rows = x.shape[0] // nchain
    chunks = [bits[i * rows:(i + 1) * rows] for i in range(nchain)]
    los = [jnp.zeros((rows, 1), jnp.int32) for _ in range(nchain)]
    for b in range(30, 12, -1):
        bit = jnp.int32(1 << b)
        cands = [lo | bit for lo in los]
        inds = [(c >= cand).astype(jnp.float32)
                for c, cand in zip(chunks, cands)]
        cnts = [jax.lax.dot_general(ind, ones_col, (((1,), (0,)), ((), ())),
                                    preferred_element_type=jnp.float32)[:, 0:1]
                for ind in inds]
        los = [jnp.where(cnt >= kf, cand, lo)
               for cnt, cand, lo in zip(cnts, cands, los)]
    lo = jnp.concatenate(los, axis=0)
    thr = jax.lax.bitcast_convert_type(lo, jnp.float32)
    ax = jax.lax.bitcast_convert_type(bits, jnp.float32)
    o_ref[...] = x * jax.nn.sigmoid(10.0 * (ax - thr))


SC_ROWS = 6656


def kernel(x):
    B, T, D = x.shape
    k = max(1, int(D * KEEP))
    R = B * T
    S = SC_ROWS
    xr = x.reshape(R, D)
    nsc = S // ROW_BLOCK
    thr_sc = _sc_thresholds(xr[:S].reshape(-1), S, D, k)
    big = pl.pallas_call(
        functools.partial(_tc_body, k),
        grid=(R // ROW_BLOCK - nsc,),
        in_specs=[pl.BlockSpec((ROW_BLOCK, D), lambda i: (i + nsc, 0))],
        out_specs=pl.BlockSpec((ROW_BLOCK, D), lambda i: (i + nsc, 0)),
        out_shape=jax.ShapeDtypeStruct((R, D), x.dtype),
    )(xr)
    out = pl.pallas_call(
        _mask_body,
        grid=(nsc,),
        in_specs=[
            pl.BlockSpec((ROW_BLOCK, D), lambda i: (i, 0)),
            pl.BlockSpec((ROW_BLOCK, 1), lambda i: (i, 0)),
            pl.BlockSpec(memory_space=pltpu.MemorySpace.HBM),
        ],
        out_specs=pl.BlockSpec((ROW_BLOCK, D), lambda i: (i, 0)),
        out_shape=jax.ShapeDtypeStruct((R, D), x.dtype),
        input_output_aliases={2: 0},
    )(xr, thr_sc.reshape(S, 1), big)
    return out.reshape(B, T, D)
